# baseline (speedup 1.0000x reference)
import jax
import jax.numpy as jnp
from jax.experimental import pallas as pl
from jax.experimental.pallas import tpu as pltpu

_C = 2000
_PAD = 8


def _gcn_body(a_ref, hs_ref, w1_ref, w2_ref, out_ref,
              a16_ref, hra_ref, hsa_ref, degs_ref):
    p = pl.program_id(0)
    i = pl.program_id(1)
    nc = pl.num_programs(1)
    last_p = pl.num_programs(0) - 1
    S, d = hs_ref.shape
    R = a16_ref.shape[0]

    @pl.when((p == 0) & (i == 0))
    def _init():
        hsa_ref[:, :d] = hs_ref[...].astype(jnp.bfloat16)
        hsa_ref[:, d:] = jnp.ones((S, _PAD), jnp.bfloat16)
        hra_ref[:, d:] = jnp.ones((R, _PAD), jnp.bfloat16)

    @pl.when(p == 0)
    def _stash_block():
        a16_ref[pl.ds(i * _C, _C), :] = a_ref[...].astype(jnp.bfloat16)

    @pl.when(p > 0)
    def _layer_step():
        blk = a16_ref[pl.ds(i * _C, _C), :]
        m_aug = jax.lax.dot_general(
            blk, hsa_ref[...], (((1,), (0,)), ((), ())),
            preferred_element_type=jnp.float32)
        deg_r = m_aug[:, d:d + 1] + 1e-8
        w1 = w1_ref[0].astype(jnp.bfloat16)
        hr = jnp.maximum(
            jax.lax.dot_general(
                m_aug[:, :d].astype(jnp.bfloat16), w1,
                (((1,), (0,)), ((), ())),
                preferred_element_type=jnp.float32),
            0.0) / deg_r

        @pl.when(p < last_p)
        def _stash_hr():
            hra_ref[pl.ds(i * _C, _C), :d] = hr.astype(jnp.bfloat16)

        @pl.when(p == last_p)
        def _emit():
            out_ref[...] = hr

        @pl.when((p < last_p) & (i == nc - 1))
        def _finish_layer():
            tT = jax.lax.dot_general(
                hra_ref[...], a16_ref[...], (((0,), (0,)), ((), ())),
                preferred_element_type=jnp.float32)

            @pl.when(p == 1)
            def _save_deg_s():
                degs_ref[...] = tT[d:d + 1, :] + 1e-8

            w2 = w2_ref[0].astype(jnp.bfloat16)
            hsT = jnp.maximum(
                jax.lax.dot_general(
                    w2, (tT[:d, :] / degs_ref[...]).astype(jnp.bfloat16),
                    (((0,), (0,)), ((), ())),
                    preferred_element_type=jnp.float32),
                0.0)
            hsa_ref[:, :d] = hsT.T.astype(jnp.bfloat16)


@jax.jit
def kernel(h_s, A_rs, r_embed, W_s2r, W_r2s):
    del r_embed
    R, S = A_rs.shape
    d = h_s.shape[1]
    L = W_s2r.shape[0]

    def a_index(p, i):
        return (jnp.where(p == 0, i, 0), 0)

    def w_index(p, i):
        return (jnp.maximum(p - 1, 0), 0, 0)

    return pl.pallas_call(
        _gcn_body,
        grid=(L + 1, R // _C),
        in_specs=[
            pl.BlockSpec((_C, S), a_index),
            pl.BlockSpec((S, d), lambda p, i: (0, 0)),
            pl.BlockSpec((1, d, d), w_index),
            pl.BlockSpec((1, d, d), w_index),
        ],
        out_specs=pl.BlockSpec((_C, d),
                               lambda p, i: (jnp.where(p == L, i, 0), 0)),
        out_shape=jax.ShapeDtypeStruct((R, d), jnp.float32),
        scratch_shapes=[
            pltpu.VMEM((R, S), jnp.bfloat16),
            pltpu.VMEM((R, d + _PAD), jnp.bfloat16),
            pltpu.VMEM((S, d + _PAD), jnp.bfloat16),
            pltpu.VMEM((1, S), jnp.float32),
        ],
    )(A_rs, h_s, W_s2r, W_r2s)


# speedup vs baseline: 1.0779x; 1.0779x over previous
import jax
import jax.numpy as jnp
from jax.experimental import pallas as pl
from jax.experimental.pallas import tpu as pltpu

_C = 2000
_PAD = 8


def _gcn_body(a_ref, hs_ref, w1_ref, w2_ref, out_ref,
              a16_ref, hra_ref, hsa_ref, degs_ref):
    l = pl.program_id(0)
    i = pl.program_id(1)
    nc = pl.num_programs(1)
    last_l = pl.num_programs(0) - 1
    S, d = hs_ref.shape
    R = a16_ref.shape[0]

    @pl.when((l == 0) & (i == 0))
    def _init():
        hsa_ref[:, :d] = hs_ref[...].astype(jnp.bfloat16)
        hsa_ref[:, d:] = jnp.ones((S, _PAD), jnp.bfloat16)
        hra_ref[:, d:] = jnp.ones((R, _PAD), jnp.bfloat16)

    @pl.when(l == 0)
    def _stash_block():
        a16_ref[pl.ds(i * _C, _C), :] = a_ref[...].astype(jnp.bfloat16)

    blk = a16_ref[pl.ds(i * _C, _C), :]
    m_aug = jax.lax.dot_general(
        blk, hsa_ref[...], (((1,), (0,)), ((), ())),
        preferred_element_type=jnp.float32)
    deg_r = m_aug[:, d:d + 1] + 1e-8
    w1 = w1_ref[0].astype(jnp.bfloat16)
    hr = jnp.maximum(
        jax.lax.dot_general(
            m_aug[:, :d].astype(jnp.bfloat16), w1, (((1,), (0,)), ((), ())),
            preferred_element_type=jnp.float32),
        0.0) / deg_r

    @pl.when(l < last_l)
    def _stash_hr():
        hra_ref[pl.ds(i * _C, _C), :d] = hr.astype(jnp.bfloat16)

    @pl.when(l == last_l)
    def _emit():
        out_ref[...] = hr

    @pl.when((l < last_l) & (i == nc - 1))
    def _finish_layer():
        tT = jax.lax.dot_general(
            hra_ref[...], a16_ref[...], (((0,), (0,)), ((), ())),
            preferred_element_type=jnp.float32)

        @pl.when(l == 0)
        def _save_deg_s():
            degs_ref[...] = tT[d:d + 1, :] + 1e-8

        w2 = w2_ref[0].astype(jnp.bfloat16)
        hsT = jnp.maximum(
            jax.lax.dot_general(
                w2, (tT[:d, :] / degs_ref[...]).astype(jnp.bfloat16),
                (((0,), (0,)), ((), ())),
                preferred_element_type=jnp.float32),
            0.0)
        hsa_ref[:, :d] = hsT.T.astype(jnp.bfloat16)


@jax.jit
def kernel(h_s, A_rs, r_embed, W_s2r, W_r2s):
    del r_embed
    R, S = A_rs.shape
    d = h_s.shape[1]
    L = W_s2r.shape[0]

    def a_index(l, i):
        return (jnp.where(l == 0, i, 0), 0)

    return pl.pallas_call(
        _gcn_body,
        grid=(L, R // _C),
        in_specs=[
            pl.BlockSpec((_C, S), a_index),
            pl.BlockSpec((S, d), lambda l, i: (0, 0)),
            pl.BlockSpec((1, d, d), lambda l, i: (l, 0, 0)),
            pl.BlockSpec((1, d, d), lambda l, i: (l, 0, 0)),
        ],
        out_specs=pl.BlockSpec((_C, d),
                               lambda l, i: (jnp.where(l == L - 1, i, 0), 0)),
        out_shape=jax.ShapeDtypeStruct((R, d), jnp.float32),
        scratch_shapes=[
            pltpu.VMEM((R, S), jnp.bfloat16),
            pltpu.VMEM((R, d + _PAD), jnp.bfloat16),
            pltpu.VMEM((S, d + _PAD), jnp.bfloat16),
            pltpu.VMEM((1, S), jnp.float32),
        ],
    )(A_rs, h_s, W_s2r, W_r2s)


# speedup vs baseline: 1.1601x; 1.0763x over previous
import jax
import jax.numpy as jnp
from jax.experimental import pallas as pl
from jax.experimental.pallas import tpu as pltpu

_C = 2000
_PAD = 8


def _gcn_body(a_ref, hs_ref, w1_ref, w2_ref, out_ref,
              a16_ref, hra_ref, hsa_ref, degs_ref):
    l = pl.program_id(0)
    i = pl.program_id(1)
    nc = pl.num_programs(1)
    last_l = pl.num_programs(0) - 1
    S, d = hs_ref.shape
    R = a16_ref.shape[0]

    @pl.when((l == 0) & (i == 0))
    def _init():
        w10 = w1_ref[0].astype(jnp.bfloat16)
        hs0 = hs_ref[...].astype(jnp.bfloat16)
        pre0 = jax.lax.dot_general(
            hs0, w10, (((1,), (0,)), ((), ())),
            preferred_element_type=jnp.float32)
        hsa_ref[:, :d] = pre0.astype(jnp.bfloat16)
        hsa_ref[:, d:] = jnp.ones((S, _PAD), jnp.bfloat16)
        hra_ref[:, d:] = jnp.ones((R, _PAD), jnp.bfloat16)

    @pl.when(l == 0)
    def _stash_block():
        a16_ref[pl.ds(i * _C, _C), :] = a_ref[...].astype(jnp.bfloat16)

    blk = a16_ref[pl.ds(i * _C, _C), :]
    m_aug = jax.lax.dot_general(
        blk, hsa_ref[...], (((1,), (0,)), ((), ())),
        preferred_element_type=jnp.float32)
    deg_r = m_aug[:, d:d + 1] + 1e-8
    hr = jnp.maximum(m_aug[:, :d], 0.0) / deg_r

    @pl.when(l < last_l)
    def _stash_hr():
        hra_ref[pl.ds(i * _C, _C), :d] = hr.astype(jnp.bfloat16)

    @pl.when(l == last_l)
    def _emit():
        out_ref[...] = hr

    @pl.when((l < last_l) & (i == nc - 1))
    def _finish_layer():
        tT = jax.lax.dot_general(
            hra_ref[...], a16_ref[...], (((0,), (0,)), ((), ())),
            preferred_element_type=jnp.float32)

        @pl.when(l == 0)
        def _save_deg_s():
            degs_ref[...] = tT[d:d + 1, :] + 1e-8

        w2 = w2_ref[l].astype(jnp.bfloat16)
        hsT = jnp.maximum(
            jax.lax.dot_general(
                w2, (tT[:d, :] / degs_ref[...]).astype(jnp.bfloat16),
                (((0,), (0,)), ((), ())),
                preferred_element_type=jnp.float32),
            0.0)
        w1n = w1_ref[l + 1].astype(jnp.bfloat16)
        preT = jax.lax.dot_general(
            w1n, hsT.astype(jnp.bfloat16), (((0,), (0,)), ((), ())),
            preferred_element_type=jnp.float32)
        hsa_ref[:, :d] = preT.T.astype(jnp.bfloat16)


@jax.jit
def kernel(h_s, A_rs, r_embed, W_s2r, W_r2s):
    del r_embed
    R, S = A_rs.shape
    d = h_s.shape[1]
    L = W_s2r.shape[0]

    def a_index(l, i):
        return (jnp.where(l == 0, i, 0), 0)

    return pl.pallas_call(
        _gcn_body,
        grid=(L, R // _C),
        in_specs=[
            pl.BlockSpec((_C, S), a_index),
            pl.BlockSpec((S, d), lambda l, i: (0, 0)),
            pl.BlockSpec((L, d, d), lambda l, i: (0, 0, 0)),
            pl.BlockSpec((L, d, d), lambda l, i: (0, 0, 0)),
        ],
        out_specs=pl.BlockSpec((_C, d),
                               lambda l, i: (jnp.where(l == L - 1, i, 0), 0)),
        out_shape=jax.ShapeDtypeStruct((R, d), jnp.float32),
        scratch_shapes=[
            pltpu.VMEM((R, S), jnp.bfloat16),
            pltpu.VMEM((R, d + _PAD), jnp.bfloat16),
            pltpu.VMEM((S, d + _PAD), jnp.bfloat16),
            pltpu.VMEM((1, S), jnp.float32),
        ],
    )(A_rs, h_s, W_s2r, W_r2s)
